# baseline (speedup 1.0000x reference)
import functools

import jax
import jax.numpy as jnp
from jax import lax
from jax.experimental import pallas as pl
from jax.experimental.pallas import tpu as pltpu
from jax.experimental.pallas import tpu_sc as plsc

BATCH = 4096
HIDDEN = 64
BASKET = 20
NW = 32
BPW = BATCH // NW
CHUNK = 64
GROWS = CHUNK * BASKET
NGD = GROWS // 128
TB = 512
LTB = 9


G = 16


def _tp_body(a_ref, o_ref):
    for g in range(G):
        blk = a_ref[:, g * 2 * TB:(g + 1) * 2 * TB].astype(jnp.bfloat16)
        ta = blk[:, :TB].T
        tb = blk[:, TB:].T
        o_ref[g * TB:(g + 1) * TB, :] = jnp.concatenate([ta, tb], axis=1)


def _transpose_pack(table_t, n_items):
    nblk = -(-n_items // (2 * TB * G))
    return pl.pallas_call(
        _tp_body,
        grid=(nblk,),
        in_specs=[pl.BlockSpec((HIDDEN, 2 * TB * G), lambda c: (0, c))],
        out_specs=pl.BlockSpec((TB * G, 2 * HIDDEN), lambda c: (c, 0)),
        out_shape=jax.ShapeDtypeStruct((nblk * TB * G, 2 * HIDDEN),
                                       jnp.bfloat16),
    )(table_t)


def _remap(idx16):
    row = ((idx16 >> (LTB + 1)) << LTB) + (idx16 & (TB - 1))
    col = ((idx16 >> LTB) & 1) << 5
    return row, col


def _sc_body(sidx_hbm, u_hbm, a_hbm, item_hbm, usr_hbm, q_out, k_out,
             sidx_v, skidx_v, scol_v, uidx_v, ucol_v, aidx_v, acol_v,
             rows_v, urows_v, arows_v, q_v, k_v, gsem, usem, asem):
    wid = lax.axis_index("s") * 2 + lax.axis_index("c")

    def chunk_body(c, carry):
        base = wid * BPW + c * CHUNK
        pltpu.sync_copy(sidx_hbm.at[pl.ds(base * BASKET, GROWS)], sidx_v)
        pltpu.sync_copy(u_hbm.at[pl.ds(base, CHUNK)], uidx_v)
        pltpu.sync_copy(a_hbm.at[pl.ds(base, CHUNK)], aidx_v)

        def smap_body(t, carry2):
            sl = pl.ds(t * 16, 16)
            row, col = _remap(sidx_v[sl])
            skidx_v[sl] = row
            scol_v[sl] = col
            return carry2

        lax.fori_loop(0, GROWS // 16, smap_body, 0)
        for t in range(CHUNK // 16):
            sl = pl.ds(t * 16, 16)
            urow, ucol = _remap(uidx_v[sl])
            uidx_v[sl] = urow
            ucol_v[sl] = ucol
            arow, acol = _remap(aidx_v[sl])
            aidx_v[sl] = arow
            acol_v[sl] = acol

        cps = []
        for r in range(NGD):
            cps.append(pltpu.async_copy(
                item_hbm.at[skidx_v.at[pl.ds(r * 128, 128)]],
                rows_v.at[pl.ds(r * 128, 128)], gsem))
        cu = pltpu.async_copy(usr_hbm.at[uidx_v], urows_v, usem)
        ca = pltpu.async_copy(item_hbm.at[aidx_v], arows_v, asem)
        for cp in cps:
            cp.wait()
        cu.wait()
        ca.wait()

        def up4(ref, b, col):
            v0 = plsc.bitcast(ref[b, pl.ds(col, 16)], jnp.bfloat16)
            v1 = plsc.bitcast(ref[b, pl.ds(col + 16, 16)], jnp.bfloat16)
            a0, b0 = plsc.unpack(v0, format=plsc.PackFormat.INTERLEAVED,
                                 preferred_element_type=jnp.float32)
            a1, b1 = plsc.unpack(v1, format=plsc.PackFormat.INTERLEAVED,
                                 preferred_element_type=jnp.float32)
            return [a0, b0, a1, b1]

        def row_body(b, carry2):
            ucol = ucol_v[pl.ds(b, 16)][0]
            acol = acol_v[pl.ds(b, 16)][0]
            c0 = scol_v[pl.ds(b * BASKET, 16)]
            c1 = scol_v[pl.ds(b * BASKET + 16, 16)]
            cols = [c0[j] if j < 16 else c1[j - 16] for j in range(BASKET)]
            accs = up4(urows_v, b, ucol)
            for j in range(BASKET):
                vs = up4(rows_v, b * BASKET + j, cols[j])
                accs = [acc + v for acc, v in zip(accs, vs)]
            ks = up4(arows_v, b, acol)
            for hs in range(HIDDEN // 16):
                q_v[b, pl.ds(hs * 16, 16)] = accs[hs]
                k_v[b, pl.ds(hs * 16, 16)] = ks[hs]
            return carry2

        lax.fori_loop(0, CHUNK, row_body, 0)
        pltpu.sync_copy(q_v, q_out.at[pl.ds(base, CHUNK)])
        pltpu.sync_copy(k_v, k_out.at[pl.ds(base, CHUNK)])
        return carry

    lax.fori_loop(0, BPW // CHUNK, chunk_body, 0)


_N_ITEM_ROWS = (-(-1000000 // (2 * TB))) * TB
_N_USR_ROWS = (-(-100000 // (2 * TB))) * TB

_sc_embed = functools.partial(
    pl.kernel,
    out_type=(
        jax.ShapeDtypeStruct((BATCH, HIDDEN), jnp.float32),
        jax.ShapeDtypeStruct((BATCH, HIDDEN), jnp.float32),
    ),
    mesh=plsc.VectorSubcoreMesh(core_axis_name="c", subcore_axis_name="s"),
    compiler_params=pltpu.CompilerParams(use_tc_tiling_on_sc=False,
                                         needs_layout_passes=False),
    scratch_types=[
        pltpu.VMEM((GROWS,), jnp.int32),
        pltpu.VMEM((GROWS,), jnp.int32),
        pltpu.VMEM((GROWS + 16,), jnp.int32),
        pltpu.VMEM((CHUNK,), jnp.int32),
        pltpu.VMEM((CHUNK + 16,), jnp.int32),
        pltpu.VMEM((CHUNK,), jnp.int32),
        pltpu.VMEM((CHUNK + 16,), jnp.int32),
        pltpu.VMEM((GROWS, HIDDEN), jnp.int32),
        pltpu.VMEM((CHUNK, HIDDEN), jnp.int32),
        pltpu.VMEM((CHUNK, HIDDEN), jnp.int32),
        pltpu.VMEM((CHUNK, HIDDEN), jnp.float32),
        pltpu.VMEM((CHUNK, HIDDEN), jnp.float32),
        pltpu.SemaphoreType.DMA,
        pltpu.SemaphoreType.DMA,
        pltpu.SemaphoreType.DMA,
    ],
)(_sc_body)


def _mm_body(q_ref, k_ref, o_ref):
    o_ref[...] = lax.dot_general(
        q_ref[...].astype(jnp.bfloat16), k_ref[...].astype(jnp.bfloat16),
        dimension_numbers=(((1,), (1,)), ((), ())),
        preferred_element_type=jnp.float32)


def _logits(q, k):
    bm, bn = 1024, 2048
    return pl.pallas_call(
        _mm_body,
        grid=(BATCH // bm, BATCH // bn),
        in_specs=[
            pl.BlockSpec((bm, HIDDEN), lambda i, j: (i, 0)),
            pl.BlockSpec((bn, HIDDEN), lambda i, j: (j, 0)),
        ],
        out_specs=pl.BlockSpec((bm, bn), lambda i, j: (i, j)),
        out_shape=jax.ShapeDtypeStruct((BATCH, BATCH), jnp.float32),
    )(q, k)


def _as_i32(t):
    return lax.bitcast_convert_type(
        t.reshape(t.shape[0], HIDDEN, 2), jnp.int32)


def kernel(U, S, A, B, item_embedding, usr_embedding):
    del B
    item_t = _as_i32(_transpose_pack(item_embedding.T, 1000000))
    usr_t = _as_i32(_transpose_pack(usr_embedding.T, 100000))
    s_last = S[:, -1, :].astype(jnp.int32).reshape(BATCH * BASKET)
    q, k = _sc_embed(s_last, U.astype(jnp.int32), A.astype(jnp.int32),
                     item_t, usr_t)
    return _logits(q, k)


# speedup vs baseline: 2.6541x; 2.6541x over previous
import functools

import jax
import jax.numpy as jnp
from jax import lax
from jax.experimental import pallas as pl
from jax.experimental.pallas import tpu as pltpu
from jax.experimental.pallas import tpu_sc as plsc

BATCH = 4096
HIDDEN = 64
BASKET = 20
NW = 32
BPW = BATCH // NW
CHUNK = 64
GROWS = CHUNK * BASKET
NGD = GROWS // 128
TB = 512
LTB = 9


G = 16


def _tp_body(a_ref, o_ref):
    for g in range(G):
        blk = a_ref[:, g * 2 * TB:(g + 1) * 2 * TB].astype(jnp.bfloat16)
        ta = blk[:, :TB].T
        tb = blk[:, TB:].T
        o_ref[g * TB:(g + 1) * TB, :] = jnp.concatenate([ta, tb], axis=1)


def _transpose_pack(table_t, n_items):
    nblk = -(-n_items // (2 * TB * G))
    return pl.pallas_call(
        _tp_body,
        grid=(nblk,),
        in_specs=[pl.BlockSpec((HIDDEN, 2 * TB * G), lambda c: (0, c))],
        out_specs=pl.BlockSpec((TB * G, 2 * HIDDEN), lambda c: (c, 0)),
        out_shape=jax.ShapeDtypeStruct((nblk * TB * G, 2 * HIDDEN),
                                       jnp.bfloat16),
    )(table_t)


def _remap(idx16):
    row = ((idx16 >> (LTB + 1)) << LTB) + (idx16 & (TB - 1))
    col = ((idx16 >> LTB) & 1) << 6
    return row, col


def _sc_body(sidx_hbm, u_hbm, a_hbm, item_hbm, usr_hbm, q_out, k_out,
             sidx_v, skidx_v, scol_v, uidx_v, ucol_v, aidx_v, acol_v,
             rows_v, urows_v, arows_v, q_v, k_v, gsem, usem, asem):
    wid = lax.axis_index("s") * 2 + lax.axis_index("c")

    def chunk_body(c, carry):
        base = wid * BPW + c * CHUNK
        pltpu.sync_copy(sidx_hbm.at[pl.ds(base * BASKET, GROWS)], sidx_v)
        pltpu.sync_copy(u_hbm.at[pl.ds(base, CHUNK)], uidx_v)
        pltpu.sync_copy(a_hbm.at[pl.ds(base, CHUNK)], aidx_v)

        def smap_body(t, carry2):
            sl = pl.ds(t * 16, 16)
            row, col = _remap(sidx_v[sl])
            skidx_v[sl] = row
            scol_v[sl] = col
            return carry2

        lax.fori_loop(0, GROWS // 16, smap_body, 0)
        for t in range(CHUNK // 16):
            sl = pl.ds(t * 16, 16)
            urow, ucol = _remap(uidx_v[sl])
            uidx_v[sl] = urow
            ucol_v[sl] = ucol
            arow, acol = _remap(aidx_v[sl])
            aidx_v[sl] = arow
            acol_v[sl] = acol

        cps = []
        for r in range(NGD):
            cps.append(pltpu.async_copy(
                item_hbm.at[skidx_v.at[pl.ds(r * 128, 128)]],
                rows_v.at[pl.ds(r * 128, 128)], gsem))
        cu = pltpu.async_copy(usr_hbm.at[uidx_v], urows_v, usem)
        ca = pltpu.async_copy(item_hbm.at[aidx_v], arows_v, asem)
        for cp in cps:
            cp.wait()
        cu.wait()
        ca.wait()

        def up4(ref, b, col):
            left = col == 0
            v0 = jnp.where(left, ref[b, pl.ds(0, 32)], ref[b, pl.ds(64, 32)])
            v1 = jnp.where(left, ref[b, pl.ds(32, 32)], ref[b, pl.ds(96, 32)])
            a0, b0 = plsc.unpack(v0, format=plsc.PackFormat.INTERLEAVED,
                                 preferred_element_type=jnp.float32)
            a1, b1 = plsc.unpack(v1, format=plsc.PackFormat.INTERLEAVED,
                                 preferred_element_type=jnp.float32)
            return [a0, b0, a1, b1]

        def row_body(b, carry2):
            ucol = ucol_v[pl.ds(b, 16)][0]
            acol = acol_v[pl.ds(b, 16)][0]
            c0 = scol_v[pl.ds(b * BASKET, 16)]
            c1 = scol_v[pl.ds(b * BASKET + 16, 16)]
            cols = [c0[j] if j < 16 else c1[j - 16] for j in range(BASKET)]
            accs = up4(urows_v, b, ucol)
            for j in range(BASKET):
                vs = up4(rows_v, b * BASKET + j, cols[j])
                accs = [acc + v for acc, v in zip(accs, vs)]
            ks = up4(arows_v, b, acol)
            for hs in range(HIDDEN // 16):
                q_v[b, pl.ds(hs * 16, 16)] = accs[hs]
                k_v[b, pl.ds(hs * 16, 16)] = ks[hs]
            return carry2

        lax.fori_loop(0, CHUNK, row_body, 0)
        pltpu.sync_copy(q_v, q_out.at[pl.ds(base, CHUNK)])
        pltpu.sync_copy(k_v, k_out.at[pl.ds(base, CHUNK)])
        return carry

    lax.fori_loop(0, BPW // CHUNK, chunk_body, 0)


_N_ITEM_ROWS = (-(-1000000 // (2 * TB))) * TB
_N_USR_ROWS = (-(-100000 // (2 * TB))) * TB

_sc_embed = functools.partial(
    pl.kernel,
    out_type=(
        jax.ShapeDtypeStruct((BATCH, HIDDEN), jnp.float32),
        jax.ShapeDtypeStruct((BATCH, HIDDEN), jnp.float32),
    ),
    mesh=plsc.VectorSubcoreMesh(core_axis_name="c", subcore_axis_name="s"),
    compiler_params=pltpu.CompilerParams(use_tc_tiling_on_sc=False,
                                         needs_layout_passes=False),
    scratch_types=[
        pltpu.VMEM((GROWS,), jnp.int32),
        pltpu.VMEM((GROWS,), jnp.int32),
        pltpu.VMEM((GROWS + 16,), jnp.int32),
        pltpu.VMEM((CHUNK,), jnp.int32),
        pltpu.VMEM((CHUNK + 16,), jnp.int32),
        pltpu.VMEM((CHUNK,), jnp.int32),
        pltpu.VMEM((CHUNK + 16,), jnp.int32),
        pltpu.VMEM((GROWS, 2 * HIDDEN), jnp.bfloat16),
        pltpu.VMEM((CHUNK, 2 * HIDDEN), jnp.bfloat16),
        pltpu.VMEM((CHUNK, 2 * HIDDEN), jnp.bfloat16),
        pltpu.VMEM((CHUNK, HIDDEN), jnp.float32),
        pltpu.VMEM((CHUNK, HIDDEN), jnp.float32),
        pltpu.SemaphoreType.DMA,
        pltpu.SemaphoreType.DMA,
        pltpu.SemaphoreType.DMA,
    ],
)(_sc_body)


def _mm_body(q_ref, k_ref, o_ref):
    o_ref[...] = lax.dot_general(
        q_ref[...].astype(jnp.bfloat16), k_ref[...].astype(jnp.bfloat16),
        dimension_numbers=(((1,), (1,)), ((), ())),
        preferred_element_type=jnp.float32)


def _logits(q, k):
    bm, bn = 1024, 2048
    return pl.pallas_call(
        _mm_body,
        grid=(BATCH // bm, BATCH // bn),
        in_specs=[
            pl.BlockSpec((bm, HIDDEN), lambda i, j: (i, 0)),
            pl.BlockSpec((bn, HIDDEN), lambda i, j: (j, 0)),
        ],
        out_specs=pl.BlockSpec((bm, bn), lambda i, j: (i, j)),
        out_shape=jax.ShapeDtypeStruct((BATCH, BATCH), jnp.float32),
    )(q, k)


def kernel(U, S, A, B, item_embedding, usr_embedding):
    del B
    item_t = _transpose_pack(item_embedding.T, 1000000)
    usr_t = _transpose_pack(usr_embedding.T, 100000)
    s_last = S[:, -1, :].astype(jnp.int32).reshape(BATCH * BASKET)
    q, k = _sc_embed(s_last, U.astype(jnp.int32), A.astype(jnp.int32),
                     item_t, usr_t)
    return _logits(q, k)


# speedup vs baseline: 5.3923x; 2.0317x over previous
import functools

import jax
import jax.numpy as jnp
from jax import lax
from jax.experimental import pallas as pl
from jax.experimental.pallas import tpu as pltpu
from jax.experimental.pallas import tpu_sc as plsc

BATCH = 4096
HIDDEN = 64
BASKET = 20
NW = 32
BPW = BATCH // NW
CHUNK = 32
GROWS = CHUNK * BASKET
NGD = GROWS // 128
TB = 512
LTB = 9


G = 16


def _tp_body(a_ref, o_ref):
    for g in range(G):
        ta = a_ref[:, g * 2 * TB:g * 2 * TB + TB].T
        tb = a_ref[:, g * 2 * TB + TB:(g + 1) * 2 * TB].T
        o_ref[g * TB:(g + 1) * TB, :] = jnp.concatenate([ta, tb], axis=1)


def _transpose_pack(table_t, n_items):
    nblk = -(-n_items // (2 * TB * G))
    return pl.pallas_call(
        _tp_body,
        grid=(nblk,),
        in_specs=[pl.BlockSpec((HIDDEN, 2 * TB * G), lambda c: (0, c))],
        out_specs=pl.BlockSpec((TB * G, 2 * HIDDEN), lambda c: (c, 0)),
        out_shape=jax.ShapeDtypeStruct((nblk * TB * G, 2 * HIDDEN),
                                       jnp.float32),
    )(table_t)


def _remap(idx16):
    row = ((idx16 >> (LTB + 1)) << LTB) + (idx16 & (TB - 1))
    col = ((idx16 >> LTB) & 1) << 6
    return row, col


def _sc_body(sidx_hbm, u_hbm, a_hbm, item_hbm, usr_hbm, q_out, k_out,
             sidx_v, skidx_v, scol_v, uidx_v, ucol_v, aidx_v, acol_v,
             rows_v, urows_v, arows_v, q_v, k_v, gsem, usem, asem):
    wid = lax.axis_index("s") * 2 + lax.axis_index("c")

    def chunk_body(c, carry):
        base = wid * BPW + c * CHUNK
        pltpu.sync_copy(sidx_hbm.at[pl.ds(base * BASKET, GROWS)], sidx_v)
        pltpu.sync_copy(u_hbm.at[pl.ds(base, CHUNK)], uidx_v)
        pltpu.sync_copy(a_hbm.at[pl.ds(base, CHUNK)], aidx_v)

        def smap_body(t, carry2):
            sl = pl.ds(t * 16, 16)
            row, col = _remap(sidx_v[sl])
            skidx_v[sl] = row
            scol_v[sl] = col
            return carry2

        lax.fori_loop(0, GROWS // 16, smap_body, 0)
        for t in range(CHUNK // 16):
            sl = pl.ds(t * 16, 16)
            urow, ucol = _remap(uidx_v[sl])
            uidx_v[sl] = urow
            ucol_v[sl] = ucol
            arow, acol = _remap(aidx_v[sl])
            aidx_v[sl] = arow
            acol_v[sl] = acol

        cps = []
        for r in range(NGD):
            cps.append(pltpu.async_copy(
                item_hbm.at[skidx_v.at[pl.ds(r * 128, 128)]],
                rows_v.at[pl.ds(r * 128, 128)], gsem))
        cu = pltpu.async_copy(usr_hbm.at[uidx_v], urows_v, usem)
        ca = pltpu.async_copy(item_hbm.at[aidx_v], arows_v, asem)
        for cp in cps:
            cp.wait()
        cu.wait()
        ca.wait()

        def row_body(b, carry2):
            ucol = ucol_v[pl.ds(b, 16)][0]
            acol = acol_v[pl.ds(b, 16)][0]
            c0 = scol_v[pl.ds(b * BASKET, 16)]
            c1 = scol_v[pl.ds(b * BASKET + 16, 16)]
            cols = [c0[j] if j < 16 else c1[j - 16] for j in range(BASKET)]
            for hs in range(HIDDEN // 16):
                h = hs * 16
                acc = urows_v[b, pl.ds(ucol + h, 16)]
                for j in range(BASKET):
                    acc = acc + rows_v[b * BASKET + j, pl.ds(cols[j] + h, 16)]
                q_v[b, pl.ds(h, 16)] = acc
                k_v[b, pl.ds(h, 16)] = arows_v[b, pl.ds(acol + h, 16)]
            return carry2

        lax.fori_loop(0, CHUNK, row_body, 0)
        pltpu.sync_copy(q_v, q_out.at[pl.ds(base, CHUNK)])
        pltpu.sync_copy(k_v, k_out.at[pl.ds(base, CHUNK)])
        return carry

    lax.fori_loop(0, BPW // CHUNK, chunk_body, 0)


_N_ITEM_ROWS = (-(-1000000 // (2 * TB))) * TB
_N_USR_ROWS = (-(-100000 // (2 * TB))) * TB

_sc_embed = functools.partial(
    pl.kernel,
    out_type=(
        jax.ShapeDtypeStruct((BATCH, HIDDEN), jnp.float32),
        jax.ShapeDtypeStruct((BATCH, HIDDEN), jnp.float32),
    ),
    mesh=plsc.VectorSubcoreMesh(core_axis_name="c", subcore_axis_name="s"),
    compiler_params=pltpu.CompilerParams(use_tc_tiling_on_sc=False),
    scratch_types=[
        pltpu.VMEM((GROWS,), jnp.int32),
        pltpu.VMEM((GROWS,), jnp.int32),
        pltpu.VMEM((GROWS + 16,), jnp.int32),
        pltpu.VMEM((CHUNK,), jnp.int32),
        pltpu.VMEM((CHUNK + 16,), jnp.int32),
        pltpu.VMEM((CHUNK,), jnp.int32),
        pltpu.VMEM((CHUNK + 16,), jnp.int32),
        pltpu.VMEM((GROWS, 2 * HIDDEN), jnp.float32),
        pltpu.VMEM((CHUNK, 2 * HIDDEN), jnp.float32),
        pltpu.VMEM((CHUNK, 2 * HIDDEN), jnp.float32),
        pltpu.VMEM((CHUNK, HIDDEN), jnp.float32),
        pltpu.VMEM((CHUNK, HIDDEN), jnp.float32),
        pltpu.SemaphoreType.DMA,
        pltpu.SemaphoreType.DMA,
        pltpu.SemaphoreType.DMA,
    ],
)(_sc_body)


def _mm_body(q_ref, k_ref, o_ref):
    o_ref[...] = lax.dot_general(
        q_ref[...], k_ref[...],
        dimension_numbers=(((1,), (1,)), ((), ())),
        preferred_element_type=jnp.float32)


def _logits(q, k):
    bm, bn = 1024, 2048
    return pl.pallas_call(
        _mm_body,
        grid=(BATCH // bm, BATCH // bn),
        in_specs=[
            pl.BlockSpec((bm, HIDDEN), lambda i, j: (i, 0)),
            pl.BlockSpec((bn, HIDDEN), lambda i, j: (j, 0)),
        ],
        out_specs=pl.BlockSpec((bm, bn), lambda i, j: (i, j)),
        out_shape=jax.ShapeDtypeStruct((BATCH, BATCH), jnp.float32),
    )(q, k)


def kernel(U, S, A, B, item_embedding, usr_embedding):
    del B
    item_t = _transpose_pack(item_embedding.T, 1000000)
    usr_t = _transpose_pack(usr_embedding.T, 100000)
    s_last = S[:, -1, :].astype(jnp.int32).reshape(BATCH * BASKET)
    q, k = _sc_embed(s_last, U.astype(jnp.int32), A.astype(jnp.int32),
                     item_t, usr_t)
    return _logits(q, k)
